# baseline (speedup 1.0000x reference)
import functools

import jax
import jax.numpy as jnp
from jax import lax
from jax.experimental import pallas as pl
from jax.experimental.pallas import tpu as pltpu
from jax.experimental.pallas import tpu_sc as plsc

_N = 16777216
_NC = 2
_NS = 16
_L = 16
_NW = _NC * _NS
_PER_W = _N // _NW
_CHUNK = 32768
_NDMA = _PER_W // _CHUNK


@functools.partial(
    pl.kernel,
    mesh=plsc.VectorSubcoreMesh(core_axis_name="c", subcore_axis_name="s"),
    out_type=jax.ShapeDtypeStruct((_N,), jnp.float32),
    scratch_types=[
        pltpu.VMEM((_CHUNK,), jnp.float32),
        pltpu.VMEM((_L,), jnp.int32),
        pltpu.VMEM((_L,), jnp.int32),
        pltpu.VMEM((_L,), jnp.int32),
        pltpu.VMEM((_L,), jnp.float32),
        pltpu.VMEM((_L,), jnp.float32),
        pltpu.SemaphoreType.DMA,
        pltpu.SemaphoreType.DMA,
    ],
)
def _sc_fill(idx_hbm, dim_hbm, n_hbm, grad_hbm, out_hbm,
             zbuf, iv, dv, nv, gvec, gtile, sem, sem2):
    c = lax.axis_index("c")
    s = lax.axis_index("s")
    wid = s * _NC + c
    base = wid * _PER_W

    scalar_copies = [
        pltpu.make_async_copy(idx_hbm, iv.at[pl.ds(0, 1)], sem2),
        pltpu.make_async_copy(dim_hbm, dv.at[pl.ds(0, 1)], sem2),
        pltpu.make_async_copy(n_hbm, nv.at[pl.ds(0, 1)], sem2),
        pltpu.make_async_copy(grad_hbm, gvec.at[pl.ds(0, 1)], sem2),
    ]
    for cp in scalar_copies:
        cp.start()

    zeros16 = jnp.zeros((_L,), jnp.float32)
    _UNROLL = 16

    def _zero_body(i, carry):
        for j in range(_UNROLL):
            zbuf[pl.ds((i * _UNROLL + j) * _L, _L)] = zeros16
        return carry

    lax.fori_loop(0, _CHUNK // (_L * _UNROLL), _zero_body, 0)

    copies = [
        pltpu.make_async_copy(
            zbuf, out_hbm.at[pl.ds(base + j * _CHUNK, _CHUNK)], sem)
        for j in range(_NDMA)
    ]
    for cp in copies:
        cp.start()

    for cp in scalar_copies:
        cp.wait()
    sidx = (iv[...][0] + dv[...][0]) % nv[...][0]
    g0 = gvec[...][0]

    for cp in copies:
        cp.wait()

    @pl.when(sidx // _PER_W == wid)
    def _():
        aligned = jnp.minimum((sidx // 8) * 8, base + _PER_W - _L)
        off = sidx - aligned
        lanes = lax.iota(jnp.int32, _L)
        gtile[...] = jnp.where(lanes == off, g0, 0.0)
        pltpu.sync_copy(gtile, out_hbm.at[pl.ds(aligned, _L)])


def kernel(grad_output, input_sizes, dim, index, out):
    del out
    idx1 = jnp.asarray(index, jnp.int32).reshape((1,))
    dim1 = jnp.asarray(dim, jnp.int32).reshape((1,))
    n1 = jnp.asarray(input_sizes, jnp.int32).reshape((1,))
    grad1 = jnp.asarray(grad_output, jnp.float32).reshape((1,))
    return _sc_fill(idx1, dim1, n1, grad1)


# speedup vs baseline: 1.0062x; 1.0062x over previous
import functools

import jax
import jax.numpy as jnp
from jax import lax
from jax.experimental import pallas as pl
from jax.experimental.pallas import tpu as pltpu
from jax.experimental.pallas import tpu_sc as plsc

_N = 16777216
_NC = 2
_NS = 16
_L = 16
_NW = _NC * _NS
_PER_W = _N // _NW
_CHUNK = 8192
_NDMA = _PER_W // _CHUNK


@functools.partial(
    pl.kernel,
    mesh=plsc.VectorSubcoreMesh(core_axis_name="c", subcore_axis_name="s"),
    out_type=jax.ShapeDtypeStruct((_N,), jnp.float32),
    scratch_types=[
        pltpu.VMEM((_CHUNK,), jnp.float32),
        pltpu.VMEM((_L,), jnp.int32),
        pltpu.VMEM((_L,), jnp.int32),
        pltpu.VMEM((_L,), jnp.int32),
        pltpu.VMEM((_L,), jnp.float32),
        pltpu.VMEM((_L,), jnp.float32),
        pltpu.SemaphoreType.DMA,
        pltpu.SemaphoreType.DMA,
    ],
)
def _sc_fill(idx_hbm, dim_hbm, n_hbm, grad_hbm, out_hbm,
             zbuf, iv, dv, nv, gvec, gtile, sem, sem2):
    c = lax.axis_index("c")
    s = lax.axis_index("s")
    wid = s * _NC + c
    base = wid * _PER_W

    scalar_copies = [
        pltpu.make_async_copy(idx_hbm, iv.at[pl.ds(0, 1)], sem2),
        pltpu.make_async_copy(dim_hbm, dv.at[pl.ds(0, 1)], sem2),
        pltpu.make_async_copy(n_hbm, nv.at[pl.ds(0, 1)], sem2),
        pltpu.make_async_copy(grad_hbm, gvec.at[pl.ds(0, 1)], sem2),
    ]
    for cp in scalar_copies:
        cp.start()

    zeros16 = jnp.zeros((_L,), jnp.float32)
    _UNROLL = 16

    def _zero_body(i, carry):
        for j in range(_UNROLL):
            zbuf[pl.ds((i * _UNROLL + j) * _L, _L)] = zeros16
        return carry

    lax.fori_loop(0, _CHUNK // (_L * _UNROLL), _zero_body, 0)

    copies = [
        pltpu.make_async_copy(
            zbuf, out_hbm.at[pl.ds(base + j * _CHUNK, _CHUNK)], sem)
        for j in range(_NDMA)
    ]
    for cp in copies:
        cp.start()

    for cp in scalar_copies:
        cp.wait()
    sidx = (iv[...][0] + dv[...][0]) % nv[...][0]
    g0 = gvec[...][0]

    for cp in copies:
        cp.wait()

    @pl.when(sidx // _PER_W == wid)
    def _():
        aligned = jnp.minimum((sidx // 8) * 8, base + _PER_W - _L)
        off = sidx - aligned
        lanes = lax.iota(jnp.int32, _L)
        gtile[...] = jnp.where(lanes == off, g0, 0.0)
        pltpu.sync_copy(gtile, out_hbm.at[pl.ds(aligned, _L)])


def kernel(grad_output, input_sizes, dim, index, out):
    del out
    idx1 = jnp.asarray(index, jnp.int32).reshape((1,))
    dim1 = jnp.asarray(dim, jnp.int32).reshape((1,))
    n1 = jnp.asarray(input_sizes, jnp.int32).reshape((1,))
    grad1 = jnp.asarray(grad_output, jnp.float32).reshape((1,))
    return _sc_fill(idx1, dim1, n1, grad1)


# speedup vs baseline: 1.0119x; 1.0057x over previous
import functools

import jax
import jax.numpy as jnp
from jax import lax
from jax.experimental import pallas as pl
from jax.experimental.pallas import tpu as pltpu
from jax.experimental.pallas import tpu_sc as plsc

_N = 16777216
_NC = 2
_NS = 16
_L = 16
_NW = _NC * _NS
_PER_W = _N // _NW
_CHUNK = 16384
_NDMA = _PER_W // _CHUNK


@functools.partial(
    pl.kernel,
    mesh=plsc.VectorSubcoreMesh(core_axis_name="c", subcore_axis_name="s"),
    out_type=jax.ShapeDtypeStruct((_N,), jnp.float32),
    scratch_types=[
        pltpu.VMEM((_CHUNK,), jnp.float32),
        pltpu.VMEM((_L,), jnp.int32),
        pltpu.VMEM((_L,), jnp.int32),
        pltpu.VMEM((_L,), jnp.int32),
        pltpu.VMEM((_L,), jnp.float32),
        pltpu.VMEM((_L,), jnp.float32),
        pltpu.SemaphoreType.DMA,
        pltpu.SemaphoreType.DMA,
    ],
)
def _sc_fill(idx_hbm, dim_hbm, n_hbm, grad_hbm, out_hbm,
             zbuf, iv, dv, nv, gvec, gtile, sem, sem2):
    c = lax.axis_index("c")
    s = lax.axis_index("s")
    wid = s * _NC + c
    base = wid * _PER_W

    scalar_copies = [
        pltpu.make_async_copy(idx_hbm, iv.at[pl.ds(0, 1)], sem2),
        pltpu.make_async_copy(dim_hbm, dv.at[pl.ds(0, 1)], sem2),
        pltpu.make_async_copy(n_hbm, nv.at[pl.ds(0, 1)], sem2),
        pltpu.make_async_copy(grad_hbm, gvec.at[pl.ds(0, 1)], sem2),
    ]
    for cp in scalar_copies:
        cp.start()

    zeros16 = jnp.zeros((_L,), jnp.float32)
    _UNROLL = 16

    def _zero_body(i, carry):
        for j in range(_UNROLL):
            zbuf[pl.ds((i * _UNROLL + j) * _L, _L)] = zeros16
        return carry

    lax.fori_loop(0, _CHUNK // (_L * _UNROLL), _zero_body, 0)

    copies = [
        pltpu.make_async_copy(
            zbuf, out_hbm.at[pl.ds(base + j * _CHUNK, _CHUNK)], sem)
        for j in range(_NDMA)
    ]
    for cp in copies:
        cp.start()

    for cp in scalar_copies:
        cp.wait()
    sidx = (iv[...][0] + dv[...][0]) % nv[...][0]
    g0 = gvec[...][0]

    for cp in copies:
        cp.wait()

    @pl.when(sidx // _PER_W == wid)
    def _():
        aligned = jnp.minimum((sidx // 8) * 8, base + _PER_W - _L)
        off = sidx - aligned
        lanes = lax.iota(jnp.int32, _L)
        gtile[...] = jnp.where(lanes == off, g0, 0.0)
        pltpu.sync_copy(gtile, out_hbm.at[pl.ds(aligned, _L)])


def kernel(grad_output, input_sizes, dim, index, out):
    del out
    idx1 = jnp.asarray(index, jnp.int32).reshape((1,))
    dim1 = jnp.asarray(dim, jnp.int32).reshape((1,))
    n1 = jnp.asarray(input_sizes, jnp.int32).reshape((1,))
    grad1 = jnp.asarray(grad_output, jnp.float32).reshape((1,))
    return _sc_fill(idx1, dim1, n1, grad1)
